# Initial kernel scaffold; baseline (speedup 1.0000x reference)
import functools

import jax
import jax.numpy as jnp
from jax import lax
from jax.experimental import pallas as pl
from jax.experimental.pallas import tpu as pltpu
from jax.experimental.pallas import tpu_sc as plsc

_VOCAB = 1000000
_D = 32
_B_TOTAL = 4096 * 200
_NC, _NS = 2, 16
_NW = _NC * _NS
_PER_W = _B_TOTAL // _NW
_C = 1024
_NCHUNK = _PER_W // _C
_GSUB = 128
_NSUB = _C // _GSUB
_SCALE = float(_D) ** 0.5


def _compute_chunk(rows, idx):
    lane = lax.iota(jnp.int32, 16)

    def group(gi, _):
        ivec = idx[pl.ds(gi * 16, 16)]
        iszero = ivec == 0
        for j in range(16):
            r = gi * 16 + j
            rows[r, pl.ds(0, 16)] = rows[r, pl.ds(0, 16)] * _SCALE
            rows[r, pl.ds(16, 16)] = rows[r, pl.ds(16, 16)] * _SCALE

        @pl.when(jnp.any(iszero))
        def _fixup():
            zero = jnp.zeros((16,), jnp.float32)
            for j in range(16):
                r = gi * 16 + j

                @pl.when(jnp.any(iszero & (lane == j)))
                def _zero_row():
                    rows[r, pl.ds(0, 16)] = zero
                    rows[r, pl.ds(16, 16)] = zero

        return 0

    lax.fori_loop(0, _C // 16, group, 0)


def _body(table, xf, out, idx0, idx1, rows0, rows1, gsem0, gsem1, wsem0, wsem1):
    c = lax.axis_index("c")
    s = lax.axis_index("s")
    wid = s * _NC + c
    base = wid * _PER_W

    idxb = (idx0, idx1)
    rowsb = (rows0, rows1)
    gsem = (gsem0, gsem1)
    wsem = (wsem0, wsem1)

    gh = {}
    wh = {}
    for g in range(_NCHUNK + 1):
        b = g % 2
        if g < _NCHUNK:
            if g >= 2:
                wh[g - 2].wait()
            pltpu.sync_copy(xf.at[pl.ds(base + g * _C, _C)], idxb[b])
            gh[g] = [
                pltpu.async_copy(
                    table.at[idxb[b].at[pl.ds(k * _GSUB, _GSUB)]],
                    rowsb[b].at[pl.ds(k * _GSUB, _GSUB), :],
                    gsem[b],
                )
                for k in range(_NSUB)
            ]
        if g >= 1:
            p = g - 1
            pb = p % 2
            for h in gh.pop(p):
                h.wait()
            _compute_chunk(rowsb[pb], idxb[pb])
            wh[p] = pltpu.async_copy(
                rowsb[pb], out.at[pl.ds(base + p * _C, _C), :], wsem[pb]
            )
    wh[_NCHUNK - 2].wait()
    wh[_NCHUNK - 1].wait()


_sc_call = functools.partial(
    pl.kernel,
    out_type=jax.ShapeDtypeStruct((_B_TOTAL, _D), jnp.float32),
    mesh=plsc.VectorSubcoreMesh(core_axis_name="c", subcore_axis_name="s"),
    scratch_types=[
        pltpu.VMEM((_C,), jnp.int32),
        pltpu.VMEM((_C,), jnp.int32),
        pltpu.VMEM((_C, _D), jnp.float32),
        pltpu.VMEM((_C, _D), jnp.float32),
        pltpu.SemaphoreType.DMA,
        pltpu.SemaphoreType.DMA,
        pltpu.SemaphoreType.DMA,
        pltpu.SemaphoreType.DMA,
    ],
)(_body)


def kernel(shared_weights, x):
    xf = x.reshape(_B_TOTAL)
    out = _sc_call(shared_weights, xf)
    return out.reshape(x.shape[0], x.shape[1], _D)


# speedup vs baseline: 1.3141x; 1.3141x over previous
import functools

import jax
import jax.numpy as jnp
from jax import lax
from jax.experimental import pallas as pl
from jax.experimental.pallas import tpu as pltpu
from jax.experimental.pallas import tpu_sc as plsc

_VOCAB = 1000000
_D = 32
_B_TOTAL = 4096 * 200
_NC, _NS = 2, 16
_NW = _NC * _NS
_PER_W = _B_TOTAL // _NW
_C = 1024
_NCHUNK = _PER_W // _C
_GSUB = 128
_NSUB = _C // _GSUB
_SCALE = float(_D) ** 0.5


def _compute_chunk(rows, idx, sbuf):
    def group(gi, _):
        ivec = idx[pl.ds(gi * 16, 16)]
        sbuf[...] = jnp.where(ivec == 0, 0.0, _SCALE).astype(jnp.float32)
        for j in range(16):
            r = gi * 16 + j
            sj = plsc.load_gather(sbuf, [jnp.full((16,), j, jnp.int32)])
            rows[r, pl.ds(0, 16)] = rows[r, pl.ds(0, 16)] * sj
            rows[r, pl.ds(16, 16)] = rows[r, pl.ds(16, 16)] * sj
        return 0

    lax.fori_loop(0, _C // 16, group, 0)


def _body(table, xf, out, idx0, idx1, rows0, rows1, sbuf, gsem0, gsem1, wsem0, wsem1):
    c = lax.axis_index("c")
    s = lax.axis_index("s")
    wid = s * _NC + c
    base = wid * _PER_W

    idxb = (idx0, idx1)
    rowsb = (rows0, rows1)
    gsem = (gsem0, gsem1)
    wsem = (wsem0, wsem1)

    gh = {}
    wh = {}
    for g in range(_NCHUNK + 1):
        b = g % 2
        if g < _NCHUNK:
            if g >= 2:
                wh[g - 2].wait()
            pltpu.sync_copy(xf.at[pl.ds(base + g * _C, _C)], idxb[b])
            gh[g] = [
                pltpu.async_copy(
                    table.at[idxb[b].at[pl.ds(k * _GSUB, _GSUB)]],
                    rowsb[b].at[pl.ds(k * _GSUB, _GSUB), :],
                    gsem[b],
                )
                for k in range(_NSUB)
            ]
        if g >= 1:
            p = g - 1
            pb = p % 2
            for h in gh.pop(p):
                h.wait()
            _compute_chunk(rowsb[pb], idxb[pb], sbuf)
            wh[p] = pltpu.async_copy(
                rowsb[pb], out.at[pl.ds(base + p * _C, _C), :], wsem[pb]
            )
    wh[_NCHUNK - 2].wait()
    wh[_NCHUNK - 1].wait()


@functools.cache
def _sc_call():
    return functools.partial(
        pl.kernel,
        out_type=jax.ShapeDtypeStruct((_B_TOTAL, _D), jnp.float32),
        compiler_params=pltpu.CompilerParams(
            needs_layout_passes=False, use_tc_tiling_on_sc=False
        ),
        mesh=plsc.VectorSubcoreMesh(
            core_axis_name="c", subcore_axis_name="s", num_cores=_NC, num_subcores=_NS
        ),
        scratch_types=[
            pltpu.VMEM((_C,), jnp.int32),
            pltpu.VMEM((_C,), jnp.int32),
            pltpu.VMEM((_C, _D), jnp.float32),
            pltpu.VMEM((_C, _D), jnp.float32),
            pltpu.VMEM((16,), jnp.float32),
            pltpu.SemaphoreType.DMA,
            pltpu.SemaphoreType.DMA,
            pltpu.SemaphoreType.DMA,
            pltpu.SemaphoreType.DMA,
        ],
    )(_body)


def kernel(shared_weights, x):
    xf = x.reshape(_B_TOTAL)
    out = _sc_call()(shared_weights, xf)
    return out.reshape(x.shape[0], x.shape[1], _D)
